# Initial kernel scaffold; baseline (speedup 1.0000x reference)
import jax
import jax.numpy as jnp
from jax.experimental import pallas as pl


def kernel(feats, codebooks):
    raise NotImplementedError("write your pallas kernel here")


# speedup vs baseline: 1.4897x; 1.4897x over previous
import jax
import jax.numpy as jnp
from jax.experimental import pallas as pl

_K = 1000
_KP = 1024
_OFF0 = 7 * _K + 1
_OFF1 = 23 * _K + 1
_D = 1024
_L = 2


def _vq_kernel(f_ref, cbp_ref, cbt_ref, tok_ref, emb_ref, pr_ref):
    toks = []
    for l in range(_L):
        fl = f_ref[:, l * _D:(l + 1) * _D]
        cbt = cbt_ref[:, l * _KP:(l + 1) * _KP]
        cbp = cbp_ref[l * _KP:(l + 1) * _KP, :]
        dots = jax.lax.dot_general(
            fl, cbt, (((1,), (0,)), ((), ())),
            preferred_element_type=jnp.float32)
        c_sq = jnp.sum(cbt * cbt, axis=0)
        col = jax.lax.broadcasted_iota(jnp.int32, dots.shape, 1)
        dist = jnp.where(col < _K,
                         c_sq[None, :] - 2.0 * dots,
                         jnp.inf)
        tok = jnp.argmin(dist, axis=1).astype(jnp.int32)
        one_hot = (col == tok[:, None]).astype(jnp.float32)
        emb = jax.lax.dot_general(
            one_hot, cbp, (((1,), (0,)), ((), ())),
            preferred_element_type=jnp.float32)
        emb_ref[:, l * _D:(l + 1) * _D] = emb
        toks.append(tok)
    tok2 = jnp.stack(toks, axis=1)
    tok_ref[...] = tok2
    colt = jax.lax.broadcasted_iota(jnp.int32, tok2.shape, 1)
    pr_ref[...] = tok2 + _OFF0 + colt * (_OFF1 - _OFF0)


@jax.jit
def kernel(feats, codebooks):
    B, T, L, D = feats.shape
    K = codebooks.shape[1]
    rows = B * T
    f2 = feats.reshape(rows, L * D)
    cb_pad = jnp.pad(codebooks, ((0, 0), (0, _KP - K), (0, 0)))
    cbp = cb_pad.reshape(L * _KP, D)
    cbt = jnp.moveaxis(cb_pad, 2, 0).reshape(D, L * _KP)
    row_tile = 400
    grid = (rows // row_tile,)
    tok2, emb2, pr2 = pl.pallas_call(
        _vq_kernel,
        grid=grid,
        in_specs=[
            pl.BlockSpec((row_tile, L * D), lambda i: (i, 0)),
            pl.BlockSpec((L * _KP, D), lambda i: (0, 0)),
            pl.BlockSpec((D, L * _KP), lambda i: (0, 0)),
        ],
        out_specs=[
            pl.BlockSpec((row_tile, L), lambda i: (i, 0)),
            pl.BlockSpec((row_tile, L * D), lambda i: (i, 0)),
            pl.BlockSpec((row_tile, L), lambda i: (i, 0)),
        ],
        out_shape=[
            jax.ShapeDtypeStruct((rows, L), jnp.int32),
            jax.ShapeDtypeStruct((rows, L * D), jnp.float32),
            jax.ShapeDtypeStruct((rows, L), jnp.int32),
        ],
    )(f2, cbp, cbt)
    tokens = tok2.reshape(B, T, L)
    embs = emb2.reshape(B, T, L, D)
    pr_tokens = pr2.reshape(B, T, L)
    return tokens, embs, pr_tokens


# speedup vs baseline: 1.5272x; 1.0252x over previous
import jax
import jax.numpy as jnp
from jax.experimental import pallas as pl
from jax.experimental.pallas import tpu as pltpu

_K = 1000
_KP = 1024
_OFF0 = 7 * _K + 1
_OFF1 = 23 * _K + 1
_D = 1024
_L = 2


def _vq_kernel(f_ref, cbp_ref, tok_ref, emb_ref, pr_ref, csq_ref):
    @pl.when(pl.program_id(0) == 0)
    def _init_csq():
        for l in range(_L):
            cbp = cbp_ref[l * _KP:(l + 1) * _KP, :]
            c_sq = jnp.sum(cbp * cbp, axis=1)
            col = jax.lax.iota(jnp.int32, _KP)
            csq_ref[l, :] = jnp.where(col < _K, c_sq, 3.0e38)

    toks = []
    for l in range(_L):
        fl = f_ref[:, l * _D:(l + 1) * _D]
        cbp = cbp_ref[l * _KP:(l + 1) * _KP, :]
        dots = jax.lax.dot_general(
            fl, cbp, (((1,), (1,)), ((), ())),
            preferred_element_type=jnp.float32)
        dist = csq_ref[l, :][None, :] - 2.0 * dots
        tok = jnp.argmin(dist, axis=1).astype(jnp.int32)
        col = jax.lax.broadcasted_iota(jnp.int32, dots.shape, 1)
        one_hot = (col == tok[:, None]).astype(jnp.float32)
        emb = jax.lax.dot_general(
            one_hot, cbp, (((1,), (0,)), ((), ())),
            preferred_element_type=jnp.float32)
        emb_ref[:, l * _D:(l + 1) * _D] = emb
        toks.append(tok)
    tok2 = jnp.stack(toks, axis=1)
    tok_ref[...] = tok2
    colt = jax.lax.broadcasted_iota(jnp.int32, tok2.shape, 1)
    pr_ref[...] = tok2 + _OFF0 + colt * (_OFF1 - _OFF0)


@jax.jit
def kernel(feats, codebooks):
    B, T, L, D = feats.shape
    K = codebooks.shape[1]
    rows = B * T
    f2 = feats.reshape(rows, L * D)
    cb_pad = jnp.pad(codebooks, ((0, 0), (0, _KP - K), (0, 0)))
    cbp = cb_pad.reshape(L * _KP, D)
    row_tile = 400
    grid = (rows // row_tile,)
    tok2, emb2, pr2 = pl.pallas_call(
        _vq_kernel,
        grid=grid,
        in_specs=[
            pl.BlockSpec((row_tile, L * D), lambda i: (i, 0)),
            pl.BlockSpec((L * _KP, D), lambda i: (0, 0)),
        ],
        out_specs=[
            pl.BlockSpec((row_tile, L), lambda i: (i, 0)),
            pl.BlockSpec((row_tile, L * D), lambda i: (i, 0)),
            pl.BlockSpec((row_tile, L), lambda i: (i, 0)),
        ],
        out_shape=[
            jax.ShapeDtypeStruct((rows, L), jnp.int32),
            jax.ShapeDtypeStruct((rows, L * D), jnp.float32),
            jax.ShapeDtypeStruct((rows, L), jnp.int32),
        ],
        scratch_shapes=[pltpu.VMEM((L, _KP), jnp.float32)],
    )(f2, cbp)
    tokens = tok2.reshape(B, T, L)
    embs = emb2.reshape(B, T, L, D)
    pr_tokens = pr2.reshape(B, T, L)
    return tokens, embs, pr_tokens


# speedup vs baseline: 1.5426x; 1.0101x over previous
import jax
import jax.numpy as jnp
from jax.experimental import pallas as pl
from jax.experimental.pallas import tpu as pltpu

_K = 1000
_OFF0 = 7 * _K + 1
_OFF1 = 23 * _K + 1
_D = 1024
_L = 2


def _vq_kernel(f_ref, cb_ref, tok_ref, emb_ref, pr_ref, csq_ref):
    @pl.when(pl.program_id(0) == 0)
    def _init_csq():
        for l in range(_L):
            cb = cb_ref[l * _K:(l + 1) * _K, :]
            csq_ref[l, :] = jnp.sum(cb * cb, axis=1)

    toks = []
    for l in range(_L):
        fl = f_ref[:, l * _D:(l + 1) * _D]
        cb = cb_ref[l * _K:(l + 1) * _K, :]
        dots = jax.lax.dot_general(
            fl, cb, (((1,), (1,)), ((), ())),
            preferred_element_type=jnp.float32)
        dist = csq_ref[l, :][None, :] - 2.0 * dots
        tok = jnp.argmin(dist, axis=1).astype(jnp.int32)
        col = jax.lax.broadcasted_iota(jnp.int32, dots.shape, 1)
        one_hot = (col == tok[:, None]).astype(jnp.float32)
        emb = jax.lax.dot_general(
            one_hot, cb, (((1,), (0,)), ((), ())),
            preferred_element_type=jnp.float32)
        emb_ref[:, l * _D:(l + 1) * _D] = emb
        toks.append(tok)
    tok2 = jnp.stack(toks, axis=1)
    tok_ref[...] = tok2
    colt = jax.lax.broadcasted_iota(jnp.int32, tok2.shape, 1)
    pr_ref[...] = tok2 + _OFF0 + colt * (_OFF1 - _OFF0)


@jax.jit
def kernel(feats, codebooks):
    B, T, L, D = feats.shape
    K = codebooks.shape[1]
    rows = B * T
    f2 = feats.reshape(rows, L * D)
    cb2 = codebooks.reshape(L * K, D)
    row_tile = 400
    grid = (rows // row_tile,)
    tok2, emb2, pr2 = pl.pallas_call(
        _vq_kernel,
        grid=grid,
        in_specs=[
            pl.BlockSpec((row_tile, L * D), lambda i: (i, 0)),
            pl.BlockSpec((L * K, D), lambda i: (0, 0)),
        ],
        out_specs=[
            pl.BlockSpec((row_tile, L), lambda i: (i, 0)),
            pl.BlockSpec((row_tile, L * D), lambda i: (i, 0)),
            pl.BlockSpec((row_tile, L), lambda i: (i, 0)),
        ],
        out_shape=[
            jax.ShapeDtypeStruct((rows, L), jnp.int32),
            jax.ShapeDtypeStruct((rows, L * D), jnp.float32),
            jax.ShapeDtypeStruct((rows, L), jnp.int32),
        ],
        scratch_shapes=[pltpu.VMEM((L, _K), jnp.float32)],
    )(f2, cb2)
    tokens = tok2.reshape(B, T, L)
    embs = emb2.reshape(B, T, L, D)
    pr_tokens = pr2.reshape(B, T, L)
    return tokens, embs, pr_tokens


# speedup vs baseline: 3.3854x; 2.1946x over previous
import jax
import jax.numpy as jnp
from jax.experimental import pallas as pl
from jax.experimental.pallas import tpu as pltpu

_K = 1000
_OFF0 = 7 * _K + 1
_OFF1 = 23 * _K + 1
_D = 1024
_L = 2
_B = 4


def _vq_kernel(f_ref, cb_ref, tok_ref, emb_ref, pr_ref, csq_ref):
    @pl.when(pl.program_id(0) == 0)
    def _init_csq():
        for l in range(_L):
            cb = cb_ref[l * _K:(l + 1) * _K, :]
            csq_ref[l, :] = jnp.sum(cb * cb, axis=1)

    tt = f_ref.shape[1]
    rows = _B * tt
    toks = []
    for l in range(_L):
        fl = f_ref[:, :, l, :].reshape(rows, _D)
        cb = cb_ref[l * _K:(l + 1) * _K, :]
        dots = jax.lax.dot_general(
            fl, cb, (((1,), (1,)), ((), ())),
            preferred_element_type=jnp.float32)
        dist = csq_ref[l, :][None, :] - 2.0 * dots
        tok = jnp.argmin(dist, axis=1).astype(jnp.int32)
        col = jax.lax.broadcasted_iota(jnp.int32, dots.shape, 1)
        one_hot = (col == tok[:, None]).astype(jnp.float32)
        emb = jax.lax.dot_general(
            one_hot, cb, (((1,), (0,)), ((), ())),
            preferred_element_type=jnp.float32)
        emb_ref[:, :, l, :] = emb.reshape(_B, tt, _D)
        toks.append(tok)
    tok2 = jnp.stack(toks, axis=1)
    tok_ref[0, :, :] = tok2
    colt = jax.lax.broadcasted_iota(jnp.int32, tok2.shape, 1)
    pr_ref[0, :, :] = tok2 + _OFF0 + colt * (_OFF1 - _OFF0)


@jax.jit
def kernel(feats, codebooks):
    B, T, L, D = feats.shape
    K = codebooks.shape[1]
    rows = B * T
    cb2 = codebooks.reshape(L * K, D)
    tt = 100
    grid = (T // tt,)
    tok2, embs, pr2 = pl.pallas_call(
        _vq_kernel,
        grid=grid,
        in_specs=[
            pl.BlockSpec((B, tt, L, D), lambda i: (0, i, 0, 0)),
            pl.BlockSpec((L * K, D), lambda i: (0, 0)),
        ],
        out_specs=[
            pl.BlockSpec((1, B * tt, L), lambda i: (i, 0, 0)),
            pl.BlockSpec((B, tt, L, D), lambda i: (0, i, 0, 0)),
            pl.BlockSpec((1, B * tt, L), lambda i: (i, 0, 0)),
        ],
        out_shape=[
            jax.ShapeDtypeStruct((T // tt, B * tt, L), jnp.int32),
            jax.ShapeDtypeStruct((B, T, L, D), jnp.float32),
            jax.ShapeDtypeStruct((T // tt, B * tt, L), jnp.int32),
        ],
        scratch_shapes=[pltpu.VMEM((L, _K), jnp.float32)],
    )(feats, cb2)
    tokens = (tok2.reshape(T // tt, B, tt, L)
              .transpose(1, 0, 2, 3).reshape(B, T, L))
    pr_tokens = (pr2.reshape(T // tt, B, tt, L)
                 .transpose(1, 0, 2, 3).reshape(B, T, L))
    return tokens, embs, pr_tokens


# speedup vs baseline: 3.5979x; 1.0628x over previous
import jax
import jax.numpy as jnp
from jax.experimental import pallas as pl
from jax.experimental.pallas import tpu as pltpu

_K = 1000
_OFF0 = 7 * _K + 1
_OFF1 = 23 * _K + 1
_D = 1024
_L = 2
_B = 4


def _vq_kernel(f_ref, cb_ref, tok_ref, emb_ref, pr_ref, csq_ref):
    @pl.when(pl.program_id(0) == 0)
    def _init_csq():
        for l in range(_L):
            cb = cb_ref[l * _K:(l + 1) * _K, :]
            csq_ref[l, :] = jnp.sum(cb * cb, axis=1)

    tt = f_ref.shape[1]
    rows = _B * tt
    toks = []
    for l in range(_L):
        fl = f_ref[:, :, l, :].reshape(rows, _D)
        cb = cb_ref[l * _K:(l + 1) * _K, :]
        dots = jax.lax.dot_general(
            fl, cb, (((1,), (1,)), ((), ())),
            preferred_element_type=jnp.float32)
        dist = csq_ref[l, :][None, :] - 2.0 * dots
        tok = jnp.argmin(dist, axis=1).astype(jnp.int32)
        col = jax.lax.broadcasted_iota(jnp.int32, dots.shape, 1)
        one_hot = (col == tok[:, None]).astype(jnp.float32)
        emb = jax.lax.dot_general(
            one_hot, cb, (((1,), (0,)), ((), ())),
            preferred_element_type=jnp.float32)
        emb_ref[:, :, l, :] = emb.reshape(_B, tt, _D)
        toks.append(tok)
    tok2 = jnp.stack(toks, axis=1)
    tok_ref[0, :, :] = tok2
    colt = jax.lax.broadcasted_iota(jnp.int32, tok2.shape, 1)
    pr_ref[0, :, :] = tok2 + _OFF0 + colt * (_OFF1 - _OFF0)


@jax.jit
def kernel(feats, codebooks):
    B, T, L, D = feats.shape
    K = codebooks.shape[1]
    rows = B * T
    cb2 = codebooks.reshape(L * K, D)
    tt = 300
    grid = (T // tt,)
    tok2, embs, pr2 = pl.pallas_call(
        _vq_kernel,
        grid=grid,
        in_specs=[
            pl.BlockSpec((B, tt, L, D), lambda i: (0, i, 0, 0)),
            pl.BlockSpec((L * K, D), lambda i: (0, 0)),
        ],
        out_specs=[
            pl.BlockSpec((1, B * tt, L), lambda i: (i, 0, 0)),
            pl.BlockSpec((B, tt, L, D), lambda i: (0, i, 0, 0)),
            pl.BlockSpec((1, B * tt, L), lambda i: (i, 0, 0)),
        ],
        out_shape=[
            jax.ShapeDtypeStruct((T // tt, B * tt, L), jnp.int32),
            jax.ShapeDtypeStruct((B, T, L, D), jnp.float32),
            jax.ShapeDtypeStruct((T // tt, B * tt, L), jnp.int32),
        ],
        scratch_shapes=[pltpu.VMEM((L, _K), jnp.float32)],
    )(feats, cb2)
    tokens = (tok2.reshape(T // tt, B, tt, L)
              .transpose(1, 0, 2, 3).reshape(B, T, L))
    pr_tokens = (pr2.reshape(T // tt, B, tt, L)
                 .transpose(1, 0, 2, 3).reshape(B, T, L))
    return tokens, embs, pr_tokens


# speedup vs baseline: 3.6222x; 1.0068x over previous
import jax
import jax.numpy as jnp
from jax.experimental import pallas as pl
from jax.experimental.pallas import tpu as pltpu

_K = 1000
_OFF0 = 7 * _K + 1
_OFF1 = 23 * _K + 1
_D = 1024
_L = 2
_B = 4


def _vq_kernel(f_ref, cb_ref, tok_ref, emb_ref, pr_ref, csq_ref):
    @pl.when(pl.program_id(0) == 0)
    def _init_csq():
        for l in range(_L):
            cb = cb_ref[l * _K:(l + 1) * _K, :]
            csq_ref[l, :] = jnp.sum(cb * cb, axis=1)

    tt = f_ref.shape[1]
    rows = _B * tt
    toks = []
    for l in range(_L):
        fl = f_ref[:, :, l, :].reshape(rows, _D)
        cb = cb_ref[l * _K:(l + 1) * _K, :]
        dots = jax.lax.dot_general(
            fl, cb, (((1,), (1,)), ((), ())),
            preferred_element_type=jnp.float32)
        dist = csq_ref[l, :][None, :] - 2.0 * dots
        tok = jnp.argmin(dist, axis=1).astype(jnp.int32)
        col = jax.lax.broadcasted_iota(jnp.int32, dots.shape, 1)
        one_hot = (col == tok[:, None]).astype(jnp.float32)
        emb = jax.lax.dot_general(
            one_hot, cb, (((1,), (0,)), ((), ())),
            preferred_element_type=jnp.float32)
        emb_ref[:, :, l, :] = emb.reshape(_B, tt, _D)
        toks.append(tok)
    tok2 = jnp.stack(toks, axis=1)
    tok_ref[0, :, :] = tok2
    colt = jax.lax.broadcasted_iota(jnp.int32, tok2.shape, 1)
    pr_ref[0, :, :] = tok2 + _OFF0 + colt * (_OFF1 - _OFF0)


@jax.jit
def kernel(feats, codebooks):
    B, T, L, D = feats.shape
    K = codebooks.shape[1]
    rows = B * T
    cb2 = codebooks.reshape(L * K, D)
    tt = 250
    grid = (T // tt,)
    tok2, embs, pr2 = pl.pallas_call(
        _vq_kernel,
        grid=grid,
        in_specs=[
            pl.BlockSpec((B, tt, L, D), lambda i: (0, i, 0, 0)),
            pl.BlockSpec((L * K, D), lambda i: (0, 0)),
        ],
        out_specs=[
            pl.BlockSpec((1, B * tt, L), lambda i: (i, 0, 0)),
            pl.BlockSpec((B, tt, L, D), lambda i: (0, i, 0, 0)),
            pl.BlockSpec((1, B * tt, L), lambda i: (i, 0, 0)),
        ],
        out_shape=[
            jax.ShapeDtypeStruct((T // tt, B * tt, L), jnp.int32),
            jax.ShapeDtypeStruct((B, T, L, D), jnp.float32),
            jax.ShapeDtypeStruct((T // tt, B * tt, L), jnp.int32),
        ],
        scratch_shapes=[pltpu.VMEM((L, _K), jnp.float32)],
    )(feats, cb2)
    tokens = (tok2.reshape(T // tt, B, tt, L)
              .transpose(1, 0, 2, 3).reshape(B, T, L))
    pr_tokens = (pr2.reshape(T // tt, B, tt, L)
                 .transpose(1, 0, 2, 3).reshape(B, T, L))
    return tokens, embs, pr_tokens
